# Initial kernel scaffold; baseline (speedup 1.0000x reference)
import jax
import jax.numpy as jnp
from jax.experimental import pallas as pl


def kernel(nodes_batch, neigh_idx, raw_features, W1, W2):
    raise NotImplementedError("write your pallas kernel here")


# speedup vs baseline: 3.3385x; 3.3385x over previous
import functools

import jax
import jax.numpy as jnp
from jax import lax
from jax.experimental import pallas as pl
from jax.experimental.pallas import tpu as pltpu
from jax.experimental.pallas import tpu_sc as plsc

_N = 100000
_S = 10
_D = 128
_B = 4096
_NC = 2
_NS = 16
_NW = _NC * _NS
_PB = _B // _NW
_L = 16
_CD = 8
_CR = _CD * _S
_GC = 128

def _expand_ids(src, dst, n_src, sem):
    del sem
    lane = lax.iota(jnp.int32, _L)

    def body(m, carry):
        v = src[pl.ds(m * _L, _L)]
        for tt in range(_S):
            j = lane + tt * _L
            pos = (j * 6554) >> 16
            slot = j - pos * _S
            g = jnp.take_along_axis(v, pos, axis=0,
                                    mode="promise_in_bounds")
            dst[pl.ds(m * _L * _S + tt * _L, _L)] = g * _S + slot
        return carry

    lax.fori_loop(0, n_src // _L, body, 0)


def _elem_gather(table_hbm, idx_v, dst, count, sem):
    def body(c, carry):
        pltpu.async_copy(table_hbm.at[idx_v.at[pl.ds(c * _GC, _GC)]],
                         dst.at[pl.ds(c * _GC, _GC)], sem).wait()
        return carry

    lax.fori_loop(0, count // _GC, body, 0)


def _agg_chunk(feat_hbm, idx_ref, idx_off, abuf, acc8, out_hbm, out_row, sem):
    pltpu.async_copy(feat_hbm.at[idx_ref.at[pl.ds(idx_off, _CR)]], abuf, sem).wait()
    for d in range(_CD):
        for v in range(_D // _L):
            acc = abuf[d * _S, pl.ds(v * _L, _L)]
            for s in range(1, _S):
                acc = acc + abuf[d * _S + s, pl.ds(v * _L, _L)]
            acc8[d, pl.ds(v * _L, _L)] = acc
    pltpu.sync_copy(acc8, out_hbm.at[pl.ds(out_row, _CD)])


@functools.partial(
    pl.kernel,
    out_type=(
        jax.ShapeDtypeStruct((_B, _D), jnp.float32),
        jax.ShapeDtypeStruct((_B, _D), jnp.float32),
        jax.ShapeDtypeStruct((_B * _S, _D), jnp.float32),
        jax.ShapeDtypeStruct((_B * _S, _D), jnp.float32),
    ),
    mesh=plsc.VectorSubcoreMesh(core_axis_name="c", subcore_axis_name="s"),
    scratch_types=[
        pltpu.VMEM((_PB,), jnp.int32),
        pltpu.VMEM((_PB * _S,), jnp.int32),
        pltpu.VMEM((_PB * _S,), jnp.int32),
        pltpu.VMEM((_PB * _S * _S,), jnp.int32),
        pltpu.VMEM((_PB * _S * _S,), jnp.int32),
        pltpu.VMEM((_PB, _D), jnp.float32),
        pltpu.VMEM((_CR, _D), jnp.float32),
        pltpu.VMEM((_CD, _D), jnp.float32),
        pltpu.SemaphoreType.DMA,
    ],
)
def _sc_gather(nodes_hbm, neigh_flat_hbm, feat_hbm,
               self_b, agg_b, self_n, agg_n,
               nb_v, e1, n1_flat, e2, n2_flat, gbuf, abuf, acc8, sem):
    wid = lax.axis_index("s") * _NC + lax.axis_index("c")
    base = wid * _PB
    nbase = wid * _PB * _S

    pltpu.sync_copy(nodes_hbm.at[pl.ds(base, _PB)], nb_v)
    _expand_ids(nb_v, e1, _PB, sem)
    _elem_gather(neigh_flat_hbm, e1, n1_flat, _PB * _S, sem)

    _expand_ids(n1_flat, e2, _PB * _S, sem)
    _elem_gather(neigh_flat_hbm, e2, n2_flat, _PB * _S * _S, sem)

    pltpu.async_copy(feat_hbm.at[nb_v], gbuf, sem).wait()
    pltpu.sync_copy(gbuf, self_b.at[pl.ds(base, _PB)])

    for c in range(_S):
        pltpu.async_copy(feat_hbm.at[n1_flat.at[pl.ds(c * _PB, _PB)]],
                         gbuf, sem).wait()
        pltpu.sync_copy(gbuf, self_n.at[pl.ds(nbase + c * _PB, _PB)])

    def aggb_body(c, carry):
        _agg_chunk(feat_hbm, n1_flat, c * _CR, abuf, acc8,
                   agg_b, base + c * _CD, sem)
        return carry

    lax.fori_loop(0, _PB * _S // _CR, aggb_body, 0)

    def aggn_body(c, carry):
        _agg_chunk(feat_hbm, n2_flat, c * _CR, abuf, acc8,
                   agg_n, nbase + c * _CD, sem)
        return carry

    lax.fori_loop(0, _PB * _S * _S // _CR, aggn_body, 0)


_PT = 512


def _tc_block(self_b_ref, agg_b_ref, self_n_ref, agg_n_ref, w1_ref, w2_ref,
              out_ref):
    dn = (((1,), (1,)), ((), ()))
    w1s = w1_ref[:, :_D]
    w1n = w1_ref[:, _D:]
    w2s = w2_ref[:, :_D]
    w2n = w2_ref[:, _D:]
    inv_s = 1.0 / _S
    h1b = lax.dot_general(self_b_ref[...], w1s, dn,
                          preferred_element_type=jnp.float32)
    h1b += lax.dot_general(agg_b_ref[...] * inv_s, w1n, dn,
                           preferred_element_type=jnp.float32)
    h1b = jnp.maximum(h1b, 0.0)
    h1n = lax.dot_general(self_n_ref[...], w1s, dn,
                          preferred_element_type=jnp.float32)
    h1n += lax.dot_general(agg_n_ref[...] * inv_s, w1n, dn,
                           preferred_element_type=jnp.float32)
    h1n = jnp.maximum(h1n, 0.0)
    agg2 = jnp.sum(h1n.reshape(_PT, _S, _D), axis=1) * inv_s
    h2 = lax.dot_general(h1b, w2s, dn, preferred_element_type=jnp.float32)
    h2 += lax.dot_general(agg2, w2n, dn, preferred_element_type=jnp.float32)
    out_ref[...] = jnp.maximum(h2, 0.0)


def _tc_forward(self_b, agg_b, self_n, agg_n, W1, W2):
    return pl.pallas_call(
        _tc_block,
        grid=(_B // _PT,),
        in_specs=[
            pl.BlockSpec((_PT, _D), lambda i: (i, 0)),
            pl.BlockSpec((_PT, _D), lambda i: (i, 0)),
            pl.BlockSpec((_PT * _S, _D), lambda i: (i, 0)),
            pl.BlockSpec((_PT * _S, _D), lambda i: (i, 0)),
            pl.BlockSpec((_D, 2 * _D), lambda i: (0, 0)),
            pl.BlockSpec((_D, 2 * _D), lambda i: (0, 0)),
        ],
        out_specs=pl.BlockSpec((_PT, _D), lambda i: (i, 0)),
        out_shape=jax.ShapeDtypeStruct((_B, _D), jnp.float32),
    )(self_b, agg_b, self_n, agg_n, W1, W2)


def kernel(nodes_batch, neigh_idx, raw_features, W1, W2):
    neigh_flat = neigh_idx.reshape(-1)
    self_b, agg_b, self_n, agg_n = _sc_gather(nodes_batch, neigh_flat,
                                              raw_features)
    return _tc_forward(self_b, agg_b, self_n, agg_n, W1, W2)


# speedup vs baseline: 4.4258x; 1.3257x over previous
import functools

import jax
import jax.numpy as jnp
from jax import lax
from jax.experimental import pallas as pl
from jax.experimental.pallas import tpu as pltpu
from jax.experimental.pallas import tpu_sc as plsc

_N = 100000
_S = 10
_D = 128
_B = 4096
_NC = 2
_NS = 16
_NW = _NC * _NS
_PB = _B // _NW
_L = 16
_CD = 8
_CR = _CD * _S
_AK = 4
_GC = 128
_NB = _B * _S

_N1_CH = _PB * _S // _GC
_N2_CH = _PB * _S * _S // _GC
_AGG_CH = (_PB * _S + _PB * _S * _S) // _CR
_AGG_N2 = _PB * _S * _S // _CR


def _expand_ids(src, src_off, dst, n_src):
    lane = lax.iota(jnp.int32, _L)

    def body(m, carry):
        v = src[pl.ds(src_off + m * _L, _L)]
        for tt in range(_S):
            j = lane + tt * _L
            pos = (j * 6554) >> 16
            slot = j - pos * _S
            g = jnp.take_along_axis(v, pos, axis=0,
                                    mode="promise_in_bounds")
            dst[pl.ds(m * _L * _S + tt * _L, _L)] = g * _S + slot
        return carry

    lax.fori_loop(0, n_src // _L, body, 0)


def _elem_gather(table_hbm, idx_v, dst, dst_off, nchunks, sem):
    descs = []
    for c in range(nchunks):
        if c >= 10:
            descs[c - 10].wait()
        descs.append(pltpu.async_copy(
            table_hbm.at[idx_v.at[pl.ds(c * _GC, _GC)]],
            dst.at[pl.ds(dst_off + c * _GC, _GC)], sem))
    for c in range(max(0, nchunks - 10), nchunks):
        descs[c].wait()


@functools.partial(
    pl.kernel,
    out_type=(
        jax.ShapeDtypeStruct((_B * (_S + 1), _D), jnp.float32),
        jax.ShapeDtypeStruct((_B * (_S + 1), _D), jnp.float32),
    ),
    mesh=plsc.VectorSubcoreMesh(core_axis_name="c", subcore_axis_name="s"),
    scratch_types=[
        pltpu.VMEM((_PB,), jnp.int32),
        pltpu.VMEM((_PB * _S,), jnp.int32),
        pltpu.VMEM((_PB * _S * _S,), jnp.int32),
        pltpu.VMEM((_PB * _S * (_S + 1),), jnp.int32),
        pltpu.VMEM((_PB, _D), jnp.float32),
        pltpu.VMEM((_PB, _D), jnp.float32),
        pltpu.VMEM((_CR, _D), jnp.float32),
        pltpu.VMEM((_CR, _D), jnp.float32),
        pltpu.VMEM((_CR, _D), jnp.float32),
        pltpu.VMEM((_CR, _D), jnp.float32),
        pltpu.VMEM((_AK * _CD, _D), jnp.float32),
        pltpu.SemaphoreType.DMA,
        pltpu.SemaphoreType.DMA,
        pltpu.SemaphoreType.DMA,
        pltpu.SemaphoreType.DMA,
        pltpu.SemaphoreType.DMA,
    ],
)
def _sc_gather(nodes_hbm, neigh_flat_hbm, feat_hbm,
               self_all, agg_all,
               nb_v, e1, e2, nall, gbuf0, gbuf1,
               abuf0, abuf1, abuf2, abuf3, accbig,
               sem_idx, sem_feat, sem_out, sem_agg, sem_aggout):
    wid = lax.axis_index("s") * _NC + lax.axis_index("c")
    base = wid * _PB
    nbase = wid * _PB * _S
    n1_off = _PB * _S * _S

    pltpu.sync_copy(nodes_hbm.at[pl.ds(base, _PB)], nb_v)
    _expand_ids(nb_v, 0, e1, _PB)
    _elem_gather(neigh_flat_hbm, e1, nall, n1_off, _N1_CH, sem_idx)
    _expand_ids(nall, n1_off, e2, _PB * _S)
    _elem_gather(neigh_flat_hbm, e2, nall, 0, _N2_CH, sem_idx)

    gbufs = (gbuf0, gbuf1)

    def _self_issue(c, buf):
        if c < _S:
            idx = nall.at[pl.ds(n1_off + c * _PB, _PB)]
        else:
            idx = nb_v
        return pltpu.async_copy(feat_hbm.at[idx], buf, sem_feat)

    def _self_row(c):
        return nbase + c * _PB if c < _S else _NB + base

    dg = {0: _self_issue(0, gbufs[0])}
    douts = {}
    for c in range(_S + 1):
        b = c % 2
        if c + 1 < _S + 1:
            if c - 1 >= 0:
                douts[c - 1].wait()
            dg[c + 1] = _self_issue(c + 1, gbufs[(c + 1) % 2])
        dg[c].wait()
        douts[c] = pltpu.async_copy(
            gbufs[b], self_all.at[pl.ds(_self_row(c), _PB)], sem_out)
    douts[_S - 1].wait()
    douts[_S].wait()

    abufs = (abuf0, abuf1, abuf2, abuf3)

    def agg_step(p, carry):
        c0 = p * _AK
        descs = [
            pltpu.async_copy(
                feat_hbm.at[nall.at[pl.ds((c0 + kk) * _CR, _CR)]],
                abufs[kk], sem_agg)
            for kk in range(_AK)
        ]
        for kk in range(_AK):
            descs[kk].wait()
            ab = abufs[kk]
            for d in range(_CD):
                for v in range(_D // _L):
                    acc = ab[d * _S, pl.ds(v * _L, _L)]
                    for s in range(1, _S):
                        acc = acc + ab[d * _S + s, pl.ds(v * _L, _L)]
                    accbig[kk * _CD + d, pl.ds(v * _L, _L)] = acc
        out_row = jnp.where(p < _AGG_N2 // _AK,
                            nbase + p * _AK * _CD,
                            _NB + base + (p - _AGG_N2 // _AK) * _AK * _CD)
        pltpu.async_copy(accbig, agg_all.at[pl.ds(out_row, _AK * _CD)],
                         sem_aggout).wait()
        return carry

    lax.fori_loop(0, _AGG_CH // _AK, agg_step, 0)


_PT = 512


def _tc_block(self_b_ref, agg_b_ref, self_n_ref, agg_n_ref, w1_ref, w2_ref,
              out_ref):
    dn = (((1,), (1,)), ((), ()))
    w1s = w1_ref[:, :_D]
    w1n = w1_ref[:, _D:]
    w2s = w2_ref[:, :_D]
    w2n = w2_ref[:, _D:]
    inv_s = 1.0 / _S
    h1b = lax.dot_general(self_b_ref[...], w1s, dn,
                          preferred_element_type=jnp.float32)
    h1b += lax.dot_general(agg_b_ref[...] * inv_s, w1n, dn,
                           preferred_element_type=jnp.float32)
    h1b = jnp.maximum(h1b, 0.0)
    h1n = lax.dot_general(self_n_ref[...], w1s, dn,
                          preferred_element_type=jnp.float32)
    h1n += lax.dot_general(agg_n_ref[...] * inv_s, w1n, dn,
                           preferred_element_type=jnp.float32)
    h1n = jnp.maximum(h1n, 0.0)
    agg2 = jnp.sum(h1n.reshape(_PT, _S, _D), axis=1) * inv_s
    h2 = lax.dot_general(h1b, w2s, dn, preferred_element_type=jnp.float32)
    h2 += lax.dot_general(agg2, w2n, dn, preferred_element_type=jnp.float32)
    out_ref[...] = jnp.maximum(h2, 0.0)


def _tc_forward(self_all, agg_all, W1, W2):
    batch_spec = pl.BlockSpec((_PT, _D), lambda i: (i + _NB // _PT, 0))
    neigh_spec = pl.BlockSpec((_PT * _S, _D), lambda i: (i, 0))
    w_spec = pl.BlockSpec((_D, 2 * _D), lambda i: (0, 0))
    return pl.pallas_call(
        _tc_block,
        grid=(_B // _PT,),
        in_specs=[batch_spec, batch_spec, neigh_spec, neigh_spec,
                  w_spec, w_spec],
        out_specs=pl.BlockSpec((_PT, _D), lambda i: (i, 0)),
        out_shape=jax.ShapeDtypeStruct((_B, _D), jnp.float32),
    )(self_all, agg_all, self_all, agg_all, W1, W2)


def kernel(nodes_batch, neigh_idx, raw_features, W1, W2):
    neigh_flat = neigh_idx.reshape(-1)
    self_all, agg_all = _sc_gather(nodes_batch, neigh_flat, raw_features)
    return _tc_forward(self_all, agg_all, W1, W2)


# speedup vs baseline: 4.4370x; 1.0025x over previous
import functools

import jax
import jax.numpy as jnp
from jax import lax
from jax.experimental import pallas as pl
from jax.experimental.pallas import tpu as pltpu
from jax.experimental.pallas import tpu_sc as plsc

_N = 100000
_S = 10
_D = 128
_B = 4096
_NC = 2
_NS = 16
_NW = _NC * _NS
_PB = _B // _NW
_L = 16
_CD = 8
_CR = _CD * _S
_AK = 4
_GC = 128
_NB = _B * _S

_N1_CH = _PB * _S // _GC
_N2_CH = _PB * _S * _S // _GC
_AGG_CH = (_PB * _S + _PB * _S * _S) // _CR
_AGG_N2 = _PB * _S * _S // _CR


def _expand_ids(src, src_off, dst, n_src):
    lane = lax.iota(jnp.int32, _L)

    def body(m, carry):
        v = src[pl.ds(src_off + m * _L, _L)]
        for tt in range(_S):
            j = lane + tt * _L
            pos = (j * 6554) >> 16
            slot = j - pos * _S
            g = jnp.take_along_axis(v, pos, axis=0,
                                    mode="promise_in_bounds")
            dst[pl.ds(m * _L * _S + tt * _L, _L)] = g * _S + slot
        return carry

    lax.fori_loop(0, n_src // _L, body, 0)


def _elem_gather(table_hbm, idx_v, dst, dst_off, nchunks, sem):
    descs = []
    for c in range(nchunks):
        if c >= 10:
            descs[c - 10].wait()
        descs.append(pltpu.async_copy(
            table_hbm.at[idx_v.at[pl.ds(c * _GC, _GC)]],
            dst.at[pl.ds(dst_off + c * _GC, _GC)], sem))
    for c in range(max(0, nchunks - 10), nchunks):
        descs[c].wait()


@functools.partial(
    pl.kernel,
    out_type=(
        jax.ShapeDtypeStruct((_B * (_S + 1), _D), jnp.float32),
        jax.ShapeDtypeStruct((_B * (_S + 1), _D), jnp.float32),
    ),
    mesh=plsc.VectorSubcoreMesh(core_axis_name="c", subcore_axis_name="s"),
    scratch_types=[
        pltpu.VMEM((_PB,), jnp.int32),
        pltpu.VMEM((_PB * _S,), jnp.int32),
        pltpu.VMEM((_PB * _S * _S,), jnp.int32),
        pltpu.VMEM((_PB * _S * (_S + 1),), jnp.int32),
        pltpu.VMEM((_PB, _D), jnp.float32),
        pltpu.VMEM((_PB, _D), jnp.float32),
        pltpu.VMEM((_CR, _D), jnp.float32),
        pltpu.VMEM((_CR, _D), jnp.float32),
        pltpu.VMEM((_CR, _D), jnp.float32),
        pltpu.VMEM((_CR, _D), jnp.float32),
        pltpu.VMEM((_AK * _CD, _D), jnp.float32),
        pltpu.SemaphoreType.DMA,
        pltpu.SemaphoreType.DMA,
        pltpu.SemaphoreType.DMA,
        pltpu.SemaphoreType.DMA,
        pltpu.SemaphoreType.DMA,
    ],
)
def _sc_gather(nodes_hbm, neigh_flat_hbm, feat_hbm,
               self_all, agg_all,
               nb_v, e1, e2, nall, gbuf0, gbuf1,
               abuf0, abuf1, abuf2, abuf3, accbig,
               sem_idx, sem_feat, sem_out, sem_agg, sem_aggout):
    wid = lax.axis_index("s") * _NC + lax.axis_index("c")
    base = wid * _PB
    nbase = wid * _PB * _S
    n1_off = _PB * _S * _S

    with jax.named_scope("ph_ids"):
        pltpu.sync_copy(nodes_hbm.at[pl.ds(base, _PB)], nb_v)
        _expand_ids(nb_v, 0, e1, _PB)
        _elem_gather(neigh_flat_hbm, e1, nall, n1_off, _N1_CH, sem_idx)
        _expand_ids(nall, n1_off, e2, _PB * _S)
        _elem_gather(neigh_flat_hbm, e2, nall, 0, _N2_CH, sem_idx)

    gbufs = (gbuf0, gbuf1)

    def _self_issue(c, buf):
        if c < _S:
            idx = nall.at[pl.ds(n1_off + c * _PB, _PB)]
        else:
            idx = nb_v
        return pltpu.async_copy(feat_hbm.at[idx], buf, sem_feat)

    def _self_row(c):
        return nbase + c * _PB if c < _S else _NB + base

    with jax.named_scope("ph_self"):
        dg = {0: _self_issue(0, gbufs[0])}
        douts = {}
        for c in range(_S + 1):
            b = c % 2
            if c + 1 < _S + 1:
                if c - 1 >= 0:
                    douts[c - 1].wait()
                dg[c + 1] = _self_issue(c + 1, gbufs[(c + 1) % 2])
            dg[c].wait()
            douts[c] = pltpu.async_copy(
                gbufs[b], self_all.at[pl.ds(_self_row(c), _PB)], sem_out)
        douts[_S - 1].wait()
        douts[_S].wait()

    abufs = (abuf0, abuf1, abuf2, abuf3)

    def agg_step(p, carry):
        c0 = p * _AK
        descs = [
            pltpu.async_copy(
                feat_hbm.at[nall.at[pl.ds((c0 + kk) * _CR, _CR)]],
                abufs[kk], sem_agg)
            for kk in range(_AK)
        ]
        for kk in range(_AK):
            descs[kk].wait()
            ab = abufs[kk]
            for d in range(_CD):
                for v in range(_D // _L):
                    acc = ab[d * _S, pl.ds(v * _L, _L)]
                    for s in range(1, _S):
                        acc = acc + ab[d * _S + s, pl.ds(v * _L, _L)]
                    accbig[kk * _CD + d, pl.ds(v * _L, _L)] = acc
        out_row = jnp.where(p < _AGG_N2 // _AK,
                            nbase + p * _AK * _CD,
                            _NB + base + (p - _AGG_N2 // _AK) * _AK * _CD)
        pltpu.async_copy(accbig, agg_all.at[pl.ds(out_row, _AK * _CD)],
                         sem_aggout).wait()
        return carry

    with jax.named_scope("ph_agg"):
        lax.fori_loop(0, _AGG_CH // _AK, agg_step, 0)


_PT = 512


def _tc_block(self_b_ref, agg_b_ref, self_n_ref, agg_n_ref, w1_ref, w2_ref,
              out_ref):
    dn = (((1,), (1,)), ((), ()))
    w1s = w1_ref[:, :_D]
    w1n = w1_ref[:, _D:]
    w2s = w2_ref[:, :_D]
    w2n = w2_ref[:, _D:]
    inv_s = 1.0 / _S
    h1b = lax.dot_general(self_b_ref[...], w1s, dn,
                          preferred_element_type=jnp.float32)
    h1b += lax.dot_general(agg_b_ref[...] * inv_s, w1n, dn,
                           preferred_element_type=jnp.float32)
    h1b = jnp.maximum(h1b, 0.0)
    h1n = lax.dot_general(self_n_ref[...], w1s, dn,
                          preferred_element_type=jnp.float32)
    h1n += lax.dot_general(agg_n_ref[...] * inv_s, w1n, dn,
                           preferred_element_type=jnp.float32)
    h1n = jnp.maximum(h1n, 0.0)
    agg2 = jnp.sum(h1n.reshape(_PT, _S, _D), axis=1) * inv_s
    h2 = lax.dot_general(h1b, w2s, dn, preferred_element_type=jnp.float32)
    h2 += lax.dot_general(agg2, w2n, dn, preferred_element_type=jnp.float32)
    out_ref[...] = jnp.maximum(h2, 0.0)


def _tc_forward(self_all, agg_all, W1, W2):
    batch_spec = pl.BlockSpec((_PT, _D), lambda i: (i + _NB // _PT, 0))
    neigh_spec = pl.BlockSpec((_PT * _S, _D), lambda i: (i, 0))
    w_spec = pl.BlockSpec((_D, 2 * _D), lambda i: (0, 0))
    return pl.pallas_call(
        _tc_block,
        grid=(_B // _PT,),
        in_specs=[batch_spec, batch_spec, neigh_spec, neigh_spec,
                  w_spec, w_spec],
        out_specs=pl.BlockSpec((_PT, _D), lambda i: (i, 0)),
        out_shape=jax.ShapeDtypeStruct((_B, _D), jnp.float32),
    )(self_all, agg_all, self_all, agg_all, W1, W2)


def kernel(nodes_batch, neigh_idx, raw_features, W1, W2):
    neigh_flat = neigh_idx.reshape(-1)
    self_all, agg_all = _sc_gather(nodes_batch, neigh_flat, raw_features)
    return _tc_forward(self_all, agg_all, W1, W2)
